# Initial kernel scaffold; baseline (speedup 1.0000x reference)
import jax
import jax.numpy as jnp
from jax.experimental import pallas as pl


def kernel(boxes, scores):
    raise NotImplementedError("write your pallas kernel here")


# speedup vs baseline: 22.2994x; 22.2994x over previous
import jax
import jax.numpy as jnp
from jax.experimental import pallas as pl
from jax.experimental.pallas import tpu as pltpu

_THRESH = 0.5
_MAX_BOX = 100
_SCORE_THRESH = 0.05
_NEG = -1e9


def _nms_body(x1_ref, y1_ref, x2_ref, y2_ref, sc_ref, out_ref, sw_ref):
    R, C = sc_ref.shape
    lin = (jax.lax.broadcasted_iota(jnp.int32, (R, C), 0) * C
           + jax.lax.broadcasted_iota(jnp.int32, (R, C), 1))
    row = jax.lax.broadcasted_iota(jnp.int32, (8, 128), 0)
    lane = jax.lax.broadcasted_iota(jnp.int32, (8, 128), 1)
    sw_ref[...] = sc_ref[...]
    x1 = x1_ref[...]
    y1 = y1_ref[...]
    x2 = x2_ref[...]
    y2 = y2_ref[...]
    areas = (x2 - x1) * (y2 - y1)

    def round_fn(i, out):
        sw = sw_ref[...]
        mx = jnp.max(sw)
        eq = sw == mx
        idx = jnp.min(jnp.where(eq, lin, jnp.int32(2147483647)))
        one = lin == idx
        onef = one.astype(jnp.float32)
        bx1 = jnp.sum(x1 * onef)
        by1 = jnp.sum(y1 * onef)
        bx2 = jnp.sum(x2 * onef)
        by2 = jnp.sum(y2 * onef)
        area_b = (bx2 - bx1) * (by2 - by1)
        iw = jnp.maximum(jnp.minimum(bx2, x2) - jnp.maximum(bx1, x1), 0.0)
        ih = jnp.maximum(jnp.minimum(by2, y2) - jnp.maximum(by1, y1), 0.0)
        inter = iw * ih
        iou = inter / jnp.maximum(area_b + areas - inter, 1e-9)
        decay = jnp.where(iou > _THRESH, 1.0 - iou, 1.0)
        sw_ref[...] = jnp.where(one, _NEG, sw * decay)
        val = jnp.where(row == 0, bx1,
              jnp.where(row == 1, by1,
              jnp.where(row == 2, bx2,
              jnp.where(row == 3, by2, mx))))
        return jnp.where(lane == i, val, out)

    out_ref[...] = jax.lax.fori_loop(0, _MAX_BOX, round_fn,
                                     jnp.zeros((8, 128), jnp.float32))


def kernel(boxes, scores):
    n = scores.shape[0]
    C = 128
    R = ((n + C - 1) // C + 7) // 8 * 8
    npad = R * C
    boxes = boxes.astype(jnp.float32)
    x1 = jnp.pad(boxes[:, 0], (0, npad - n)).reshape(R, C)
    y1 = jnp.pad(boxes[:, 1], (0, npad - n)).reshape(R, C)
    x2 = jnp.pad(boxes[:, 2], (0, npad - n)).reshape(R, C)
    y2 = jnp.pad(boxes[:, 3], (0, npad - n)).reshape(R, C)
    sc = jnp.pad(scores.astype(jnp.float32), (0, npad - n),
                 constant_values=_NEG).reshape(R, C)
    out = pl.pallas_call(
        _nms_body,
        out_shape=jax.ShapeDtypeStruct((8, 128), jnp.float32),
        scratch_shapes=[pltpu.VMEM((R, C), jnp.float32)],
    )(x1, y1, x2, y2, sc)
    sel = out[:5, :_MAX_BOX]
    s = sel[4]
    valid = (s >= _SCORE_THRESH).astype(boxes.dtype)
    kept_boxes = sel[:4].T * valid[:, None]
    kept_scores = s * valid
    return jnp.concatenate([kept_boxes, kept_scores[:, None]], axis=1)
